# baseline (speedup 1.0000x reference)
import jax
import jax.numpy as jnp
from jax.experimental import pallas as pl

N_MODELS = 8
BATCH = 1024
IN_DIM = 1024
OUT_DIM = 1024


def _mm_kernel(x_ref, w_ref, b_ref, o_ref):
    o_ref[0] = (
        jnp.dot(x_ref[0], w_ref[0], preferred_element_type=jnp.float32)
        + b_ref[0]
    )


def kernel(inputs, weight, bias):
    out = pl.pallas_call(
        _mm_kernel,
        grid=(N_MODELS,),
        in_specs=[
            pl.BlockSpec((1, BATCH, IN_DIM), lambda i: (i, 0, 0)),
            pl.BlockSpec((1, IN_DIM, OUT_DIM), lambda i: (i, 0, 0)),
            pl.BlockSpec((1, 1, OUT_DIM), lambda i: (i, 0, 0)),
        ],
        out_specs=pl.BlockSpec((1, BATCH, OUT_DIM), lambda i: (i, 0, 0)),
        out_shape=jax.ShapeDtypeStruct((N_MODELS, BATCH, OUT_DIM), jnp.float32),
    )(inputs, weight, bias.reshape(N_MODELS, 1, OUT_DIM))
    return tuple(out[i] for i in range(N_MODELS))
